# baseline (speedup 1.0000x reference)
import jax
import jax.numpy as jnp
from jax.experimental import pallas as pl
from jax.experimental.pallas import tpu as pltpu

N = 50000
NCLS = 5
NSEL = 5000
RB, LB = 392, 128
P = RB * LB
NPAD = P - N
NSELT = NSEL + NPAD


def _body(label_ref, attn_ref, cols_hbm, out_ref, cols_vmem, dma_sem):
    lbl = label_ref[0]
    row = attn_ref[...]

    cols_copy = pltpu.make_async_copy(cols_hbm, cols_vmem, dma_sem)
    cols_copy.start()

    i0 = jax.lax.broadcasted_iota(jnp.int32, (RB, LB), 0)
    i1 = jax.lax.broadcasted_iota(jnp.int32, (RB, LB), 1)
    idx = i0 * LB + i1
    valid = idx < N

    xi = jax.lax.bitcast_convert_type(row, jnp.int32)

    def _narrow(lo, hi, t1, t2, t3, p1, p2, p3):
        new_lo = jnp.where(p3, t3, jnp.where(p2, t2, jnp.where(p1, t1, lo)))
        new_hi = jnp.where(p1, jnp.where(p2, jnp.where(p3, hi, t3), t2), t1)
        return new_lo, new_hi

    def _cuts(lo, hi):
        d = hi - lo
        return lo + d // 4, lo + d // 2, hi - d // 4

    BIAS = 16384
    hi16 = ((xi >> 15) - BIAS).astype(jnp.int16)
    lo16 = ((xi & 0x7FFF) - BIAS).astype(jnp.int16)

    def _cnt16(mask_arr):
        part = jnp.sum(mask_arr.astype(jnp.int16), axis=0)
        return jnp.sum(part.astype(jnp.int32))

    def vstep1(_, st):
        tlo, thi, blo, bhi = st
        t1, t2, t3 = _cuts(tlo, thi)
        u1, u2, u3 = _cuts(blo, bhi)
        tp1 = _cnt16(hi16 >= t1.astype(jnp.int16)) >= NSELT
        tp2 = _cnt16(hi16 >= t2.astype(jnp.int16)) >= NSELT
        tp3 = _cnt16(hi16 >= t3.astype(jnp.int16)) >= NSELT
        bp1 = _cnt16(hi16 <= u1.astype(jnp.int16)) < NSEL
        bp2 = _cnt16(hi16 <= u2.astype(jnp.int16)) < NSEL
        bp3 = _cnt16(hi16 <= u3.astype(jnp.int16)) < NSEL
        tlo, thi = _narrow(tlo, thi, t1, t2, t3, tp1, tp2, tp3)
        blo, bhi = _narrow(blo, bhi, u1, u2, u3, bp1, bp2, bp3)
        return tlo, thi, blo, bhi

    LOW1, HIGH1 = jnp.int32(-BIAS - 1), jnp.int32(BIAS + 1)
    K1t, _, _, K1b = jax.lax.fori_loop(
        0, 9, vstep1, (LOW1, HIGH1, LOW1, HIGH1))

    eq_t = hi16 == K1t.astype(jnp.int16)
    eq_b = hi16 == K1b.astype(jnp.int16)
    rem_t = NSELT - _cnt16(hi16 > K1t.astype(jnp.int16))
    rem_b = NSEL - _cnt16(hi16 < K1b.astype(jnp.int16))

    def vstep2(_, st):
        tlo, thi, blo, bhi = st
        t1, t2, t3 = _cuts(tlo, thi)
        u1, u2, u3 = _cuts(blo, bhi)
        tp1 = _cnt16(eq_t & (lo16 >= t1.astype(jnp.int16))) >= rem_t
        tp2 = _cnt16(eq_t & (lo16 >= t2.astype(jnp.int16))) >= rem_t
        tp3 = _cnt16(eq_t & (lo16 >= t3.astype(jnp.int16))) >= rem_t
        bp1 = _cnt16(eq_b & (lo16 <= u1.astype(jnp.int16))) < rem_b
        bp2 = _cnt16(eq_b & (lo16 <= u2.astype(jnp.int16))) < rem_b
        bp3 = _cnt16(eq_b & (lo16 <= u3.astype(jnp.int16))) < rem_b
        tlo, thi = _narrow(tlo, thi, t1, t2, t3, tp1, tp2, tp3)
        blo, bhi = _narrow(blo, bhi, u1, u2, u3, bp1, bp2, bp3)
        return tlo, thi, blo, bhi

    K2t, _, _, K2b = jax.lax.fori_loop(
        0, 9, vstep2, (LOW1, HIGH1, LOW1, HIGH1))

    K = ((K1t + BIAS) << 15) | (K2t + BIAS)
    K2 = ((K1b + BIAS) << 15) | (K2b + BIAS)

    gt = xi > K
    eq = xi == K
    need = NSELT - jnp.sum(gt.astype(jnp.int32))
    cnt_eq = jnp.sum(eq.astype(jnp.int32))
    lt = xi < K2
    eq2 = xi == K2
    need2 = NSEL - jnp.sum(lt.astype(jnp.int32))
    cnt_eq2 = jnp.sum(eq2.astype(jnp.int32))

    def _no_ties():
        return jnp.int32(P), jnp.int32(P)

    def _resolve_ties():
        def istep(_, st):
            tlo, thi, blo, bhi = st
            t1, t2, t3 = _cuts(tlo, thi)
            u1, u2, u3 = _cuts(blo, bhi)
            tp1 = jnp.sum((eq & (idx < t1)).astype(jnp.int32)) < need
            tp2 = jnp.sum((eq & (idx < t2)).astype(jnp.int32)) < need
            tp3 = jnp.sum((eq & (idx < t3)).astype(jnp.int32)) < need
            bp1 = jnp.sum((eq2 & (idx < u1)).astype(jnp.int32)) < need2
            bp2 = jnp.sum((eq2 & (idx < u2)).astype(jnp.int32)) < need2
            bp3 = jnp.sum((eq2 & (idx < u3)).astype(jnp.int32)) < need2
            tlo, thi = _narrow(tlo, thi, t1, t2, t3, tp1, tp2, tp3)
            blo, bhi = _narrow(blo, bhi, u1, u2, u3, bp1, bp2, bp3)
            return tlo, thi, blo, bhi

        _, cut_, _, cut2_ = jax.lax.fori_loop(
            0, 10, istep,
            (jnp.int32(0), jnp.int32(P), jnp.int32(0), jnp.int32(P)))
        return cut_, cut2_

    cut, cut2 = jax.lax.cond(
        (cnt_eq == need) & (cnt_eq2 == need2), _no_ties, _resolve_ties)

    top_mask = (gt & valid) | (eq & (idx < cut))
    bot_mask = lt | (eq2 & (idx < cut2))

    cols_copy.wait()
    c = [cols_vmem[j] for j in range(NCLS + 1)]
    m = c[0]
    for j in range(1, NCLS + 1):
        m = jnp.maximum(m, c[j])
    s = jnp.zeros_like(m)
    for j in range(NCLS + 1):
        s = s + jnp.exp(c[j] - m)
    lse = m + jnp.log(s)
    a = jnp.zeros_like(m)
    for j in range(NCLS):
        a = a + jnp.where(lbl == j, c[j], 0.0)
    tval = lse - a
    bval = lse - c[NCLS]

    sum_top = jnp.sum(jnp.where(top_mask, tval, 0.0))
    sum_bot = jnp.sum(jnp.where(bot_mask, bval, 0.0))
    total = (sum_top + sum_bot) / (2.0 * NSEL)
    out_ref[...] = jnp.broadcast_to(total, (1, 1))


def kernel(ref_logits, attn, label):
    cols = jnp.transpose(ref_logits)
    cols = jnp.pad(cols, ((0, 0), (0, NPAD))).reshape(NCLS + 1, RB, LB)
    lbl = jnp.asarray(label, dtype=jnp.int32).reshape(1)
    row_p = jnp.pad(attn[lbl[0]], (0, NPAD), constant_values=2.0)
    row_p = row_p.reshape(RB, LB)
    grid_spec = pltpu.PrefetchScalarGridSpec(
        num_scalar_prefetch=1,
        grid=(1,),
        in_specs=[
            pl.BlockSpec((RB, LB), lambda i, lbl_ref: (0, 0)),
            pl.BlockSpec(memory_space=pl.ANY),
        ],
        out_specs=pl.BlockSpec((1, 1), lambda i, lbl_ref: (0, 0)),
        scratch_shapes=[
            pltpu.VMEM((NCLS + 1, RB, LB), jnp.float32),
            pltpu.SemaphoreType.DMA,
        ],
    )
    out = pl.pallas_call(
        _body,
        grid_spec=grid_spec,
        out_shape=jax.ShapeDtypeStruct((1, 1), jnp.float32),
    )(lbl, row_p, cols)
    return out[0, 0]


# speedup vs baseline: 1.4855x; 1.4855x over previous
import jax
import jax.numpy as jnp
from jax.experimental import pallas as pl
from jax.experimental.pallas import tpu as pltpu

N = 50000
NCLS = 5
NSEL = 5000
RB, LB = 392, 128
P = RB * LB
NPAD = P - N
NSELT = NSEL + NPAD


def _body(label_ref, attn_ref, cols_hbm, out_ref, cols_vmem, dma_sem):
    lbl = label_ref[0]
    row = attn_ref[...]

    cols_copy = pltpu.make_async_copy(cols_hbm, cols_vmem, dma_sem)
    cols_copy.start()

    i0 = jax.lax.broadcasted_iota(jnp.int32, (RB, LB), 0)
    i1 = jax.lax.broadcasted_iota(jnp.int32, (RB, LB), 1)
    idx = i0 * LB + i1
    valid = idx < N

    xi = jax.lax.bitcast_convert_type(row, jnp.int32)

    def _narrow(lo, hi, t1, t2, t3, p1, p2, p3):
        new_lo = jnp.where(p3, t3, jnp.where(p2, t2, jnp.where(p1, t1, lo)))
        new_hi = jnp.where(p1, jnp.where(p2, jnp.where(p3, hi, t3), t2), t1)
        return new_lo, new_hi

    def _cuts(lo, hi):
        d = hi - lo
        return lo + d // 4, lo + d // 2, hi - d // 4

    def vstep(_, st):
        tlo, thi, blo, bhi = st
        t1, t2, t3 = _cuts(tlo, thi)
        u1, u2, u3 = _cuts(blo, bhi)
        tp1 = jnp.sum((xi >= t1).astype(jnp.int32)) >= NSELT
        tp2 = jnp.sum((xi >= t2).astype(jnp.int32)) >= NSELT
        tp3 = jnp.sum((xi >= t3).astype(jnp.int32)) >= NSELT
        bp1 = jnp.sum((xi <= u1).astype(jnp.int32)) < NSEL
        bp2 = jnp.sum((xi <= u2).astype(jnp.int32)) < NSEL
        bp3 = jnp.sum((xi <= u3).astype(jnp.int32)) < NSEL
        tlo, thi = _narrow(tlo, thi, t1, t2, t3, tp1, tp2, tp3)
        blo, bhi = _narrow(blo, bhi, u1, u2, u3, bp1, bp2, bp3)
        return tlo, thi, blo, bhi

    HIv = jnp.int32(0x3F800000)
    st = (jnp.int32(-1), HIv, jnp.int32(-1), HIv)
    for _i in range(17):
        st = vstep(_i, st)
    K, _, _, K2 = st

    gt = xi > K
    eq = xi == K
    need = NSELT - jnp.sum(gt.astype(jnp.int32))
    cnt_eq = jnp.sum(eq.astype(jnp.int32))
    lt = xi < K2
    eq2 = xi == K2
    need2 = NSEL - jnp.sum(lt.astype(jnp.int32))
    cnt_eq2 = jnp.sum(eq2.astype(jnp.int32))

    def _no_ties():
        return jnp.int32(P), jnp.int32(P)

    def _resolve_ties():
        def istep(_, st):
            tlo, thi, blo, bhi = st
            t1, t2, t3 = _cuts(tlo, thi)
            u1, u2, u3 = _cuts(blo, bhi)
            tp1 = jnp.sum((eq & (idx < t1)).astype(jnp.int32)) < need
            tp2 = jnp.sum((eq & (idx < t2)).astype(jnp.int32)) < need
            tp3 = jnp.sum((eq & (idx < t3)).astype(jnp.int32)) < need
            bp1 = jnp.sum((eq2 & (idx < u1)).astype(jnp.int32)) < need2
            bp2 = jnp.sum((eq2 & (idx < u2)).astype(jnp.int32)) < need2
            bp3 = jnp.sum((eq2 & (idx < u3)).astype(jnp.int32)) < need2
            tlo, thi = _narrow(tlo, thi, t1, t2, t3, tp1, tp2, tp3)
            blo, bhi = _narrow(blo, bhi, u1, u2, u3, bp1, bp2, bp3)
            return tlo, thi, blo, bhi

        _, cut_, _, cut2_ = jax.lax.fori_loop(
            0, 10, istep,
            (jnp.int32(0), jnp.int32(P), jnp.int32(0), jnp.int32(P)))
        return cut_, cut2_

    cut, cut2 = jax.lax.cond(
        (cnt_eq == need) & (cnt_eq2 == need2), _no_ties, _resolve_ties)

    top_mask = (gt & valid) | (eq & (idx < cut))
    bot_mask = lt | (eq2 & (idx < cut2))

    cols_copy.wait()
    c = [cols_vmem[j] for j in range(NCLS + 1)]
    m = c[0]
    for j in range(1, NCLS + 1):
        m = jnp.maximum(m, c[j])
    s = jnp.zeros_like(m)
    for j in range(NCLS + 1):
        s = s + jnp.exp(c[j] - m)
    lse = m + jnp.log(s)
    a = jnp.zeros_like(m)
    for j in range(NCLS):
        a = a + jnp.where(lbl == j, c[j], 0.0)
    tval = lse - a
    bval = lse - c[NCLS]

    sum_top = jnp.sum(jnp.where(top_mask, tval, 0.0))
    sum_bot = jnp.sum(jnp.where(bot_mask, bval, 0.0))
    total = (sum_top + sum_bot) / (2.0 * NSEL)
    out_ref[...] = jnp.broadcast_to(total, (1, 1))


def kernel(ref_logits, attn, label):
    cols = jnp.transpose(ref_logits)
    cols = jnp.pad(cols, ((0, 0), (0, NPAD))).reshape(NCLS + 1, RB, LB)
    lbl = jnp.asarray(label, dtype=jnp.int32).reshape(1)
    row_p = jnp.pad(attn[lbl[0]], (0, NPAD), constant_values=2.0)
    row_p = row_p.reshape(RB, LB)
    grid_spec = pltpu.PrefetchScalarGridSpec(
        num_scalar_prefetch=1,
        grid=(1,),
        in_specs=[
            pl.BlockSpec((RB, LB), lambda i, lbl_ref: (0, 0)),
            pl.BlockSpec(memory_space=pl.ANY),
        ],
        out_specs=pl.BlockSpec((1, 1), lambda i, lbl_ref: (0, 0)),
        scratch_shapes=[
            pltpu.VMEM((NCLS + 1, RB, LB), jnp.float32),
            pltpu.SemaphoreType.DMA,
        ],
    )
    out = pl.pallas_call(
        _body,
        grid_spec=grid_spec,
        out_shape=jax.ShapeDtypeStruct((1, 1), jnp.float32),
    )(lbl, row_p, cols)
    return out[0, 0]


# speedup vs baseline: 1.5875x; 1.0687x over previous
import jax
import jax.numpy as jnp
from jax.experimental import pallas as pl
from jax.experimental.pallas import tpu as pltpu

N = 50000
NCLS = 5
NSEL = 5000
RB, LB = 392, 128
P = RB * LB
NPAD = P - N
NSELT = NSEL + NPAD


def _body(label_ref, attn_ref, cols_hbm, out_ref, cols_vmem, dma_sem):
    lbl = label_ref[0]
    row = attn_ref[...]

    cols_copy = pltpu.make_async_copy(cols_hbm, cols_vmem, dma_sem)
    cols_copy.start()

    i0 = jax.lax.broadcasted_iota(jnp.int32, (RB, LB), 0)
    i1 = jax.lax.broadcasted_iota(jnp.int32, (RB, LB), 1)
    idx = i0 * LB + i1
    valid = idx < N

    xi = jax.lax.bitcast_convert_type(row, jnp.int32)

    def _narrow(lo, hi, t1, t2, t3, p1, p2, p3):
        new_lo = jnp.where(p3, t3, jnp.where(p2, t2, jnp.where(p1, t1, lo)))
        new_hi = jnp.where(p1, jnp.where(p2, jnp.where(p3, hi, t3), t2), t1)
        return new_lo, new_hi

    def _cuts(lo, hi):
        d = hi - lo
        return lo + d // 4, lo + d // 2, hi - d // 4

    def vstep(_, st):
        tlo, thi, blo, bhi = st
        t1, t2, t3 = _cuts(tlo, thi)
        u1, u2, u3 = _cuts(blo, bhi)
        tp1 = jnp.sum((xi >= t1).astype(jnp.int32)) >= NSELT
        tp2 = jnp.sum((xi >= t2).astype(jnp.int32)) >= NSELT
        tp3 = jnp.sum((xi >= t3).astype(jnp.int32)) >= NSELT
        bp1 = jnp.sum((xi <= u1).astype(jnp.int32)) < NSEL
        bp2 = jnp.sum((xi <= u2).astype(jnp.int32)) < NSEL
        bp3 = jnp.sum((xi <= u3).astype(jnp.int32)) < NSEL
        tlo, thi = _narrow(tlo, thi, t1, t2, t3, tp1, tp2, tp3)
        blo, bhi = _narrow(blo, bhi, u1, u2, u3, bp1, bp2, bp3)
        return tlo, thi, blo, bhi

    HIv = jnp.int32(0x3F800000)
    K, _, _, K2 = jax.lax.fori_loop(
        0, 17, vstep, (jnp.int32(-1), HIv, jnp.int32(-1), HIv))

    gt = xi > K
    eq = xi == K
    need = NSELT - jnp.sum(gt.astype(jnp.int32))
    cnt_eq = jnp.sum(eq.astype(jnp.int32))
    lt = xi < K2
    eq2 = xi == K2
    need2 = NSEL - jnp.sum(lt.astype(jnp.int32))
    cnt_eq2 = jnp.sum(eq2.astype(jnp.int32))

    def _no_ties():
        return jnp.int32(P), jnp.int32(P)

    def _resolve_ties():
        def istep(_, st):
            tlo, thi, blo, bhi = st
            t1, t2, t3 = _cuts(tlo, thi)
            u1, u2, u3 = _cuts(blo, bhi)
            tp1 = jnp.sum((eq & (idx < t1)).astype(jnp.int32)) < need
            tp2 = jnp.sum((eq & (idx < t2)).astype(jnp.int32)) < need
            tp3 = jnp.sum((eq & (idx < t3)).astype(jnp.int32)) < need
            bp1 = jnp.sum((eq2 & (idx < u1)).astype(jnp.int32)) < need2
            bp2 = jnp.sum((eq2 & (idx < u2)).astype(jnp.int32)) < need2
            bp3 = jnp.sum((eq2 & (idx < u3)).astype(jnp.int32)) < need2
            tlo, thi = _narrow(tlo, thi, t1, t2, t3, tp1, tp2, tp3)
            blo, bhi = _narrow(blo, bhi, u1, u2, u3, bp1, bp2, bp3)
            return tlo, thi, blo, bhi

        _, cut_, _, cut2_ = jax.lax.fori_loop(
            0, 10, istep,
            (jnp.int32(0), jnp.int32(P), jnp.int32(0), jnp.int32(P)))
        return cut_, cut2_

    cut, cut2 = jax.lax.cond(
        (cnt_eq == need) & (cnt_eq2 == need2), _no_ties, _resolve_ties)

    top_mask = (gt & valid) | (eq & (idx < cut))
    bot_mask = lt | (eq2 & (idx < cut2))

    cols_copy.wait()
    c = [cols_vmem[j] for j in range(NCLS + 1)]
    m = c[0]
    for j in range(1, NCLS + 1):
        m = jnp.maximum(m, c[j])
    s = jnp.zeros_like(m)
    for j in range(NCLS + 1):
        s = s + jnp.exp(c[j] - m)
    lse = m + jnp.log(s)
    a = jnp.zeros_like(m)
    for j in range(NCLS):
        a = a + jnp.where(lbl == j, c[j], 0.0)
    tval = lse - a
    bval = lse - c[NCLS]

    sum_top = jnp.sum(jnp.where(top_mask, tval, 0.0))
    sum_bot = jnp.sum(jnp.where(bot_mask, bval, 0.0))
    total = (sum_top + sum_bot) / (2.0 * NSEL)
    out_ref[...] = jnp.broadcast_to(total, (1, 1))


def kernel(ref_logits, attn, label):
    cols = jnp.transpose(ref_logits)
    cols = jnp.pad(cols, ((0, 0), (0, NPAD))).reshape(NCLS + 1, RB, LB)
    lbl = jnp.asarray(label, dtype=jnp.int32).reshape(1)
    row_p = jnp.pad(attn[lbl[0]], (0, NPAD), constant_values=2.0)
    row_p = row_p.reshape(RB, LB)
    grid_spec = pltpu.PrefetchScalarGridSpec(
        num_scalar_prefetch=1,
        grid=(1,),
        in_specs=[
            pl.BlockSpec((RB, LB), lambda i, lbl_ref: (0, 0)),
            pl.BlockSpec(memory_space=pl.ANY),
        ],
        out_specs=pl.BlockSpec((1, 1), lambda i, lbl_ref: (0, 0)),
        scratch_shapes=[
            pltpu.VMEM((NCLS + 1, RB, LB), jnp.float32),
            pltpu.SemaphoreType.DMA,
        ],
    )
    out = pl.pallas_call(
        _body,
        grid_spec=grid_spec,
        out_shape=jax.ShapeDtypeStruct((1, 1), jnp.float32),
    )(lbl, row_p, cols)
    return out[0, 0]


# speedup vs baseline: 1.6290x; 1.0261x over previous
import jax
import jax.numpy as jnp
from jax.experimental import pallas as pl
from jax.experimental.pallas import tpu as pltpu

N = 50000
NCLS = 5
NSEL = 5000
RB, LB = 392, 128
P = RB * LB
NPAD = P - N
NSELT = NSEL + NPAD


def _body(label_ref, attn_ref, cols_hbm, out_ref, cols_vmem, dma_sem):
    lbl = label_ref[0]
    row = attn_ref[...]

    cols_copy = pltpu.make_async_copy(cols_hbm, cols_vmem, dma_sem)
    cols_copy.start()

    i0 = jax.lax.broadcasted_iota(jnp.int32, (RB, LB), 0)
    i1 = jax.lax.broadcasted_iota(jnp.int32, (RB, LB), 1)
    idx = i0 * LB + i1
    valid = idx < N

    xi = jax.lax.bitcast_convert_type(row, jnp.int32)

    def _narrow(lo, hi, t1, t2, t3, p1, p2, p3):
        new_lo = jnp.where(p3, t3, jnp.where(p2, t2, jnp.where(p1, t1, lo)))
        new_hi = jnp.where(p1, jnp.where(p2, jnp.where(p3, hi, t3), t2), t1)
        return new_lo, new_hi

    def _cuts(lo, hi):
        d = hi - lo
        return lo + d // 4, lo + d // 2, hi - d // 4

    def vstep(_, st):
        tlo, thi, blo, bhi = st
        t1, t2, t3 = _cuts(tlo, thi)
        u1, u2, u3 = _cuts(blo, bhi)
        tp1 = jnp.sum((xi >= t1).astype(jnp.int32)) >= NSELT
        tp2 = jnp.sum((xi >= t2).astype(jnp.int32)) >= NSELT
        tp3 = jnp.sum((xi >= t3).astype(jnp.int32)) >= NSELT
        bp1 = jnp.sum((xi <= u1).astype(jnp.int32)) < NSEL
        bp2 = jnp.sum((xi <= u2).astype(jnp.int32)) < NSEL
        bp3 = jnp.sum((xi <= u3).astype(jnp.int32)) < NSEL
        tlo, thi = _narrow(tlo, thi, t1, t2, t3, tp1, tp2, tp3)
        blo, bhi = _narrow(blo, bhi, u1, u2, u3, bp1, bp2, bp3)
        return tlo, thi, blo, bhi

    HIv = jnp.int32(0x3F800000)
    K, _, _, K2 = jax.lax.fori_loop(
        0, 16, vstep, (jnp.int32(-1), HIv, jnp.int32(-1), HIv))

    gt = xi > K
    eq = xi == K
    need = NSELT - jnp.sum(gt.astype(jnp.int32))
    cnt_eq = jnp.sum(eq.astype(jnp.int32))
    lt = xi < K2
    eq2 = xi == K2
    need2 = NSEL - jnp.sum(lt.astype(jnp.int32))
    cnt_eq2 = jnp.sum(eq2.astype(jnp.int32))

    def _no_ties():
        return jnp.int32(P), jnp.int32(P)

    def _resolve_ties():
        def istep(_, st):
            tlo, thi, blo, bhi = st
            t1, t2, t3 = _cuts(tlo, thi)
            u1, u2, u3 = _cuts(blo, bhi)
            tp1 = jnp.sum((eq & (idx < t1)).astype(jnp.int32)) < need
            tp2 = jnp.sum((eq & (idx < t2)).astype(jnp.int32)) < need
            tp3 = jnp.sum((eq & (idx < t3)).astype(jnp.int32)) < need
            bp1 = jnp.sum((eq2 & (idx < u1)).astype(jnp.int32)) < need2
            bp2 = jnp.sum((eq2 & (idx < u2)).astype(jnp.int32)) < need2
            bp3 = jnp.sum((eq2 & (idx < u3)).astype(jnp.int32)) < need2
            tlo, thi = _narrow(tlo, thi, t1, t2, t3, tp1, tp2, tp3)
            blo, bhi = _narrow(blo, bhi, u1, u2, u3, bp1, bp2, bp3)
            return tlo, thi, blo, bhi

        _, cut_, _, cut2_ = jax.lax.fori_loop(
            0, 8, istep,
            (jnp.int32(0), jnp.int32(P), jnp.int32(0), jnp.int32(P)))
        return cut_, cut2_

    cut, cut2 = jax.lax.cond(
        (cnt_eq == need) & (cnt_eq2 == need2), _no_ties, _resolve_ties)

    top_mask = (gt & valid) | (eq & (idx < cut))
    bot_mask = lt | (eq2 & (idx < cut2))

    cols_copy.wait()
    c = [cols_vmem[j] for j in range(NCLS + 1)]
    m = c[0]
    for j in range(1, NCLS + 1):
        m = jnp.maximum(m, c[j])
    s = jnp.zeros_like(m)
    for j in range(NCLS + 1):
        s = s + jnp.exp(c[j] - m)
    lse = m + jnp.log(s)
    a = jnp.zeros_like(m)
    for j in range(NCLS):
        a = a + jnp.where(lbl == j, c[j], 0.0)
    tval = lse - a
    bval = lse - c[NCLS]

    sum_top = jnp.sum(jnp.where(top_mask, tval, 0.0))
    sum_bot = jnp.sum(jnp.where(bot_mask, bval, 0.0))
    total = (sum_top + sum_bot) / (2.0 * NSEL)
    out_ref[...] = jnp.broadcast_to(total, (1, 1))


def kernel(ref_logits, attn, label):
    cols = jnp.transpose(ref_logits)
    cols = jnp.pad(cols, ((0, 0), (0, NPAD))).reshape(NCLS + 1, RB, LB)
    lbl = jnp.asarray(label, dtype=jnp.int32).reshape(1)
    row_p = jnp.pad(attn[lbl[0]], (0, NPAD), constant_values=2.0)
    row_p = row_p.reshape(RB, LB)
    grid_spec = pltpu.PrefetchScalarGridSpec(
        num_scalar_prefetch=1,
        grid=(1,),
        in_specs=[
            pl.BlockSpec((RB, LB), lambda i, lbl_ref: (0, 0)),
            pl.BlockSpec(memory_space=pl.ANY),
        ],
        out_specs=pl.BlockSpec((1, 1), lambda i, lbl_ref: (0, 0)),
        scratch_shapes=[
            pltpu.VMEM((NCLS + 1, RB, LB), jnp.float32),
            pltpu.SemaphoreType.DMA,
        ],
    )
    out = pl.pallas_call(
        _body,
        grid_spec=grid_spec,
        out_shape=jax.ShapeDtypeStruct((1, 1), jnp.float32),
    )(lbl, row_p, cols)
    return out[0, 0]
